# Initial kernel scaffold; baseline (speedup 1.0000x reference)
import jax
import jax.numpy as jnp
from jax.experimental import pallas as pl


def kernel(x, edge_index, edge_type, W_rel_0, b_0, W_self_0, gamma_0, beta_0, W_rel_1, b_1, W_self_1, gamma_1, beta_1, W_rel_2, b_2, W_self_2, gamma_2, beta_2):
    raise NotImplementedError("write your pallas kernel here")


# speedup vs baseline: 3.8905x; 3.8905x over previous
import functools

import jax
import jax.numpy as jnp
from jax import lax
from jax.experimental import pallas as pl
from jax.experimental.pallas import tpu as pltpu
from jax.experimental.pallas import tpu_sc as plsc

NN = 10000
EE = 640000
RR = 7
HID = 128

NW = 32
CH = 128
KC = 160
IB = 32
EPAD = NW * KC * CH
ACC_ROWS = 10240
RPT = ACC_ROWS // 16

BN = 1000
NB = NN // BN

def _sc_body(t_hbm, gidx_hbm, dst_hbm, out_hbm, idx_v, didx_v, rows_v,
             acc_sh):
    c = lax.axis_index("c")
    s = lax.axis_index("s")
    wid = s * 2 + c

    zv = jnp.zeros((16,), jnp.float32)

    def _zrow(i, carry):
        for t in range(HID // 16):
            rows_v[i, pl.ds(t * 16, 16)] = zv
        return carry

    lax.fori_loop(0, CH, _zrow, 0)
    for kk in range(RPT // CH):
        pltpu.sync_copy(rows_v, acc_sh.at[pl.ds(s * RPT + kk * CH, CH)])
    plsc.subcore_barrier()

    def _iblock(ib, carry):
        pltpu.sync_copy(gidx_hbm.at[wid, pl.ds(ib * IB, IB)], idx_v)
        pltpu.sync_copy(dst_hbm.at[wid, pl.ds(ib * IB, IB)], didx_v)

        def _chunk(j, carry2):
            pltpu.sync_copy(t_hbm.at[idx_v.at[j]], rows_v)
            pltpu.sync_copy(rows_v, acc_sh.at[didx_v.at[j]], add=True)
            return carry2

        lax.fori_loop(0, IB, _chunk, 0)
        return carry

    lax.fori_loop(0, KC // IB, _iblock, 0)

    plsc.subcore_barrier()
    pltpu.sync_copy(acc_sh.at[pl.ds(s * RPT, RPT)],
                    out_hbm.at[c, pl.ds(s * RPT, RPT)])


@functools.lru_cache(maxsize=None)
def _get_sc_aggregate():
    mesh = plsc.VectorSubcoreMesh(core_axis_name="c", subcore_axis_name="s")
    return pl.kernel(
        _sc_body,
        out_type=jax.ShapeDtypeStruct((2, ACC_ROWS, HID), jnp.float32),
        mesh=mesh,
        scratch_types=[
            pltpu.VMEM((IB, CH), jnp.int32),
            pltpu.VMEM((IB, CH), jnp.int32),
            pltpu.VMEM((CH, HID), jnp.float32),
            pltpu.VMEM_SHARED((ACC_ROWS, HID), jnp.float32),
        ],
    )


def _sc_aggregate(t2, gidx3, dst3):
    return _get_sc_aggregate()(t2, gidx3, dst3)


def _mm_body(h_ref, w2_ref, ws_ref, t_ref, s_ref):
    h = h_ref[...]
    t_ref[...] = jnp.dot(h, w2_ref[...], preferred_element_type=jnp.float32)
    s_ref[...] = jnp.dot(h, ws_ref[...], preferred_element_type=jnp.float32)


_matmuls = pl.pallas_call(
    _mm_body,
    grid=(NB,),
    in_specs=[
        pl.BlockSpec((BN, HID), lambda j: (j, 0)),
        pl.BlockSpec((HID, RR * HID), lambda j: (0, 0)),
        pl.BlockSpec((HID, HID), lambda j: (0, 0)),
    ],
    out_specs=[
        pl.BlockSpec((BN, RR * HID), lambda j: (j, 0)),
        pl.BlockSpec((BN, HID), lambda j: (j, 0)),
    ],
    out_shape=[
        jax.ShapeDtypeStruct((NN, RR * HID), jnp.float32),
        jax.ShapeDtypeStruct((NN, HID), jnp.float32),
    ],
)


def _norm_body(p_ref, s_ref, b_ref, g_ref, be_ref, o_ref, pre_buf, stats):
    ph = pl.program_id(0)
    j = pl.program_id(1)

    @pl.when(ph == 0)
    def _():
        pre = p_ref[0] + p_ref[1] + s_ref[...] + b_ref[...]
        pre_buf[pl.ds(j * BN, BN)] = pre

        @pl.when(j == 0)
        def _():
            stats[...] = jnp.zeros_like(stats)

        stats[0:1, :] += jnp.sum(pre, axis=0, keepdims=True)
        stats[1:2, :] += jnp.sum(pre * pre, axis=0, keepdims=True)
        o_ref[...] = pre

    @pl.when(ph == 1)
    def _():
        @pl.when(j == 0)
        def _():
            mean = stats[0:1, :] * (1.0 / NN)
            var = stats[1:2, :] * (1.0 / NN) - mean * mean
            scale = g_ref[...] * lax.rsqrt(var + 1e-5)
            shift = be_ref[...] - mean * scale
            stats[0:1, :] = scale
            stats[1:2, :] = shift

        pre = pre_buf[pl.ds(j * BN, BN)]
        o_ref[...] = jnp.maximum(pre * stats[0:1, :] + stats[1:2, :], 0.0)


_norm = pl.pallas_call(
    _norm_body,
    grid=(2, NB),
    in_specs=[
        pl.BlockSpec((2, BN, HID), lambda p, j: (0, j, 0)),
        pl.BlockSpec((BN, HID), lambda p, j: (j, 0)),
        pl.BlockSpec((1, HID), lambda p, j: (0, 0)),
        pl.BlockSpec((1, HID), lambda p, j: (0, 0)),
        pl.BlockSpec((1, HID), lambda p, j: (0, 0)),
    ],
    out_specs=pl.BlockSpec((BN, HID), lambda p, j: (j, 0)),
    out_shape=jax.ShapeDtypeStruct((NN, HID), jnp.float32),
    scratch_shapes=[
        pltpu.VMEM((NN, HID), jnp.float32),
        pltpu.VMEM((2, HID), jnp.float32),
    ],
)


def _regroup_w(w_rel, din):
    w3 = w_rel.reshape(RR, din, HID)
    if din < HID:
        w3 = jnp.pad(w3, ((0, 0), (0, HID - din), (0, 0)))
    return w3.transpose(1, 0, 2).reshape(HID, RR * HID)


def _pad_cols(a, din):
    if din < HID:
        return jnp.pad(a, ((0, 0), (0, HID - din)))
    return a


def kernel(x, edge_index, edge_type, W_rel_0, b_0, W_self_0, gamma_0, beta_0,
           W_rel_1, b_1, W_self_1, gamma_1, beta_1, W_rel_2, b_2, W_self_2,
           gamma_2, beta_2):
    src = edge_index[0]
    dst = edge_index[1]
    gidx = src * RR + edge_type
    npad = EPAD - EE
    gidx3 = jnp.concatenate(
        [gidx, jnp.zeros((npad,), jnp.int32)]).reshape(NW, KC, CH)
    dst3 = jnp.concatenate(
        [dst, jnp.full((npad,), NN, jnp.int32)]).reshape(NW, KC, CH)

    h = _pad_cols(x, x.shape[1])
    dins = [x.shape[1], HID, HID]
    params = [
        (W_rel_0, b_0, W_self_0, gamma_0, beta_0),
        (W_rel_1, b_1, W_self_1, gamma_1, beta_1),
        (W_rel_2, b_2, W_self_2, gamma_2, beta_2),
    ]
    outs = []
    for l, (wr, b, ws, g, be) in enumerate(params):
        w2 = _regroup_w(wr, dins[l])
        wsp = jnp.pad(ws, ((0, HID - dins[l]), (0, 0))) if dins[l] < HID else ws
        t2, s = _matmuls(h, w2, wsp)
        part = _sc_aggregate(t2.reshape(NN * RR, HID), gidx3, dst3)
        h = _norm(part, s, b.reshape(1, HID), g.reshape(1, HID),
                  be.reshape(1, HID))
        outs.append(h)
    return jnp.concatenate(outs, axis=-1)


# speedup vs baseline: 4.1344x; 1.0627x over previous
import functools

import jax
import jax.numpy as jnp
from jax import lax
from jax.experimental import pallas as pl
from jax.experimental.pallas import tpu as pltpu
from jax.experimental.pallas import tpu_sc as plsc

NN = 10000
EE = 640000
RR = 7
HID = 128

NW = 32
CH = 128
KC = 160
IB = 32
EPAD = NW * KC * CH
ACC_ROWS = 10240
RPT = ACC_ROWS // 16

BN = 1000
NB = NN // BN

def _sc_body(t_hbm, gidx_hbm, dst_hbm, out_hbm, idx_v, didx_v, rows_a, rows_b,
             acc_sh, gsem_a, gsem_b, ssem_a, ssem_b):
    c = lax.axis_index("c")
    s = lax.axis_index("s")
    wid = s * 2 + c

    zv = jnp.zeros((16,), jnp.float32)

    def _zrow(i, carry):
        for t in range(HID // 16):
            rows_a[i, pl.ds(t * 16, 16)] = zv
        return carry

    lax.fori_loop(0, CH, _zrow, 0)
    for kk in range(RPT // CH):
        pltpu.sync_copy(rows_a, acc_sh.at[pl.ds(s * RPT + kk * CH, CH)])
    plsc.subcore_barrier()

    def _iblock(ib, carry):
        pltpu.sync_copy(gidx_hbm.at[wid, pl.ds(ib * IB, IB)], idx_v)
        pltpu.sync_copy(dst_hbm.at[wid, pl.ds(ib * IB, IB)], didx_v)
        pltpu.async_copy(t_hbm.at[idx_v.at[0]], rows_a, gsem_a)
        pltpu.async_copy(t_hbm.at[idx_v.at[1]], rows_b, gsem_b)

        def _pair(i, c2):
            j = 2 * i
            pltpu.make_async_copy(t_hbm.at[idx_v.at[j]], rows_a, gsem_a).wait()
            pltpu.async_copy(rows_a, acc_sh.at[didx_v.at[j]], ssem_a, add=True)
            pltpu.make_async_copy(t_hbm.at[idx_v.at[j + 1]], rows_b,
                                  gsem_b).wait()
            pltpu.async_copy(rows_b, acc_sh.at[didx_v.at[j + 1]], ssem_b,
                             add=True)

            @pl.when(j + 2 < IB)
            def _():
                pltpu.make_async_copy(rows_a, acc_sh.at[didx_v.at[j]],
                                      ssem_a).wait()
                pltpu.async_copy(t_hbm.at[idx_v.at[j + 2]], rows_a, gsem_a)
                pltpu.make_async_copy(rows_b, acc_sh.at[didx_v.at[j + 1]],
                                      ssem_b).wait()
                pltpu.async_copy(t_hbm.at[idx_v.at[j + 3]], rows_b, gsem_b)

            return c2

        lax.fori_loop(0, IB // 2, _pair, 0)
        pltpu.make_async_copy(rows_a, acc_sh.at[didx_v.at[0]], ssem_a).wait()
        pltpu.make_async_copy(rows_b, acc_sh.at[didx_v.at[0]], ssem_b).wait()
        return carry

    lax.fori_loop(0, KC // IB, _iblock, 0)

    plsc.subcore_barrier()
    pltpu.sync_copy(acc_sh.at[pl.ds(s * RPT, RPT)],
                    out_hbm.at[c, pl.ds(s * RPT, RPT)])


@functools.lru_cache(maxsize=None)
def _get_sc_aggregate():
    mesh = plsc.VectorSubcoreMesh(core_axis_name="c", subcore_axis_name="s")
    return pl.kernel(
        _sc_body,
        out_type=jax.ShapeDtypeStruct((2, ACC_ROWS, HID), jnp.float32),
        mesh=mesh,
        scratch_types=[
            pltpu.VMEM((IB, CH), jnp.int32),
            pltpu.VMEM((IB, CH), jnp.int32),
            pltpu.VMEM((CH, HID), jnp.float32),
            pltpu.VMEM((CH, HID), jnp.float32),
            pltpu.VMEM_SHARED((ACC_ROWS, HID), jnp.float32),
            pltpu.SemaphoreType.DMA,
            pltpu.SemaphoreType.DMA,
            pltpu.SemaphoreType.DMA,
            pltpu.SemaphoreType.DMA,
        ],
    )


def _sc_aggregate(t2, gidx3, dst3):
    return _get_sc_aggregate()(t2, gidx3, dst3)


def _mm_body(h_ref, w2_ref, ws_ref, t_ref, s_ref):
    h = h_ref[...]
    t_ref[...] = jnp.dot(h, w2_ref[...], preferred_element_type=jnp.float32,
                         precision=lax.Precision.HIGHEST)
    s_ref[...] = jnp.dot(h, ws_ref[...], preferred_element_type=jnp.float32,
                         precision=lax.Precision.HIGHEST)


_matmuls = pl.pallas_call(
    _mm_body,
    grid=(NB,),
    in_specs=[
        pl.BlockSpec((BN, HID), lambda j: (j, 0)),
        pl.BlockSpec((HID, RR * HID), lambda j: (0, 0)),
        pl.BlockSpec((HID, HID), lambda j: (0, 0)),
    ],
    out_specs=[
        pl.BlockSpec((BN, RR * HID), lambda j: (j, 0)),
        pl.BlockSpec((BN, HID), lambda j: (j, 0)),
    ],
    out_shape=[
        jax.ShapeDtypeStruct((NN, RR * HID), jnp.float32),
        jax.ShapeDtypeStruct((NN, HID), jnp.float32),
    ],
)


def _norm_body(p_ref, s_ref, b_ref, g_ref, be_ref, o_ref, pre_buf, stats):
    ph = pl.program_id(0)
    j = pl.program_id(1)

    @pl.when(ph == 0)
    def _():
        pre = p_ref[0] + p_ref[1] + s_ref[...] + b_ref[...]
        pre_buf[pl.ds(j * BN, BN)] = pre

        @pl.when(j == 0)
        def _():
            stats[...] = jnp.zeros_like(stats)

        stats[0:1, :] += jnp.sum(pre, axis=0, keepdims=True)
        stats[1:2, :] += jnp.sum(pre * pre, axis=0, keepdims=True)
        o_ref[...] = pre

    @pl.when(ph == 1)
    def _():
        @pl.when(j == 0)
        def _():
            mean = stats[0:1, :] * (1.0 / NN)
            var = stats[1:2, :] * (1.0 / NN) - mean * mean
            scale = g_ref[...] * lax.rsqrt(var + 1e-5)
            shift = be_ref[...] - mean * scale
            stats[0:1, :] = scale
            stats[1:2, :] = shift

        pre = pre_buf[pl.ds(j * BN, BN)]
        o_ref[...] = jnp.maximum(pre * stats[0:1, :] + stats[1:2, :], 0.0)


_norm = pl.pallas_call(
    _norm_body,
    grid=(2, NB),
    in_specs=[
        pl.BlockSpec((2, BN, HID), lambda p, j: (0, j, 0)),
        pl.BlockSpec((BN, HID), lambda p, j: (j, 0)),
        pl.BlockSpec((1, HID), lambda p, j: (0, 0)),
        pl.BlockSpec((1, HID), lambda p, j: (0, 0)),
        pl.BlockSpec((1, HID), lambda p, j: (0, 0)),
    ],
    out_specs=pl.BlockSpec((BN, HID), lambda p, j: (j, 0)),
    out_shape=jax.ShapeDtypeStruct((NN, HID), jnp.float32),
    scratch_shapes=[
        pltpu.VMEM((NN, HID), jnp.float32),
        pltpu.VMEM((2, HID), jnp.float32),
    ],
)


def _regroup_w(w_rel, din):
    w3 = w_rel.reshape(RR, din, HID)
    if din < HID:
        w3 = jnp.pad(w3, ((0, 0), (0, HID - din), (0, 0)))
    return w3.transpose(1, 0, 2).reshape(HID, RR * HID)


def _pad_cols(a, din):
    if din < HID:
        return jnp.pad(a, ((0, 0), (0, HID - din)))
    return a


def kernel(x, edge_index, edge_type, W_rel_0, b_0, W_self_0, gamma_0, beta_0,
           W_rel_1, b_1, W_self_1, gamma_1, beta_1, W_rel_2, b_2, W_self_2,
           gamma_2, beta_2):
    src = edge_index[0]
    dst = edge_index[1]
    gidx = src * RR + edge_type
    npad = EPAD - EE
    gidx3 = jnp.concatenate(
        [gidx, jnp.zeros((npad,), jnp.int32)]).reshape(NW, KC, CH)
    dst3 = jnp.concatenate(
        [dst, jnp.full((npad,), NN, jnp.int32)]).reshape(NW, KC, CH)

    h = _pad_cols(x, x.shape[1])
    dins = [x.shape[1], HID, HID]
    params = [
        (W_rel_0, b_0, W_self_0, gamma_0, beta_0),
        (W_rel_1, b_1, W_self_1, gamma_1, beta_1),
        (W_rel_2, b_2, W_self_2, gamma_2, beta_2),
    ]
    outs = []
    for l, (wr, b, ws, g, be) in enumerate(params):
        w2 = _regroup_w(wr, dins[l])
        wsp = jnp.pad(ws, ((0, HID - dins[l]), (0, 0))) if dins[l] < HID else ws
        t2, s = _matmuls(h, w2, wsp)
        part = _sc_aggregate(t2.reshape(NN * RR, HID), gidx3, dst3)
        h = _norm(part, s, b.reshape(1, HID), g.reshape(1, HID),
                  be.reshape(1, HID))
        outs.append(h)
    return jnp.concatenate(outs, axis=-1)


# speedup vs baseline: 4.4430x; 1.0746x over previous
import functools

import jax
import jax.numpy as jnp
from jax import lax
from jax.experimental import pallas as pl
from jax.experimental.pallas import tpu as pltpu
from jax.experimental.pallas import tpu_sc as plsc

NN = 10000
EE = 640000
RR = 7
HID = 128

NW = 32
CH = 128
IB = 16
KF = 240
KS = 80
FAST_CORE = 0
TOTCH = 16 * (KF + KS)
EPAD = TOTCH * CH
ACC_ROWS = 10240
RPT = ACC_ROWS // 16

BN = 1000
NB = NN // BN

def _sc_body(t_hbm, gidx_hbm, dst_hbm, out_hbm, idx_v, didx_v, rows_a, rows_b,
             acc_sh, gsem_a, gsem_b, ssem_a, ssem_b):
    c = lax.axis_index("c")
    s = lax.axis_index("s")
    on_fast = c == FAST_CORE
    base_chunk = jnp.where(on_fast, s * KF, 16 * KF + s * KS)
    nblk = jnp.where(on_fast, KF // IB, KS // IB)

    zv = jnp.zeros((16,), jnp.float32)

    def _zrow(i, carry):
        for t in range(HID // 16):
            rows_a[i, pl.ds(t * 16, 16)] = zv
        return carry

    lax.fori_loop(0, CH, _zrow, 0)
    for kk in range(RPT // CH):
        pltpu.sync_copy(rows_a, acc_sh.at[pl.ds(s * RPT + kk * CH, CH)])
    plsc.subcore_barrier()

    def _iblock(ib, carry):
        pltpu.sync_copy(gidx_hbm.at[pl.ds(base_chunk + ib * IB, IB)], idx_v)
        pltpu.sync_copy(dst_hbm.at[pl.ds(base_chunk + ib * IB, IB)], didx_v)
        pltpu.async_copy(t_hbm.at[idx_v.at[0]], rows_a, gsem_a)
        pltpu.async_copy(t_hbm.at[idx_v.at[1]], rows_b, gsem_b)

        def _pair(i, c2):
            j = 2 * i
            pltpu.make_async_copy(t_hbm.at[idx_v.at[j]], rows_a, gsem_a).wait()
            pltpu.async_copy(rows_a, acc_sh.at[didx_v.at[j]], ssem_a, add=True)
            pltpu.make_async_copy(t_hbm.at[idx_v.at[j + 1]], rows_b,
                                  gsem_b).wait()
            pltpu.async_copy(rows_b, acc_sh.at[didx_v.at[j + 1]], ssem_b,
                             add=True)

            @pl.when(j + 2 < IB)
            def _():
                pltpu.make_async_copy(rows_a, acc_sh.at[didx_v.at[j]],
                                      ssem_a).wait()
                pltpu.async_copy(t_hbm.at[idx_v.at[j + 2]], rows_a, gsem_a)
                pltpu.make_async_copy(rows_b, acc_sh.at[didx_v.at[j + 1]],
                                      ssem_b).wait()
                pltpu.async_copy(t_hbm.at[idx_v.at[j + 3]], rows_b, gsem_b)

            return c2

        lax.fori_loop(0, IB // 2, _pair, 0)
        pltpu.make_async_copy(rows_a, acc_sh.at[didx_v.at[0]], ssem_a).wait()
        pltpu.make_async_copy(rows_b, acc_sh.at[didx_v.at[0]], ssem_b).wait()
        return carry

    lax.fori_loop(0, nblk, _iblock, 0)

    plsc.subcore_barrier()
    pltpu.sync_copy(acc_sh.at[pl.ds(s * RPT, RPT)],
                    out_hbm.at[c, pl.ds(s * RPT, RPT)])


@functools.lru_cache(maxsize=None)
def _get_sc_aggregate():
    mesh = plsc.VectorSubcoreMesh(core_axis_name="c", subcore_axis_name="s")
    return pl.kernel(
        _sc_body,
        out_type=jax.ShapeDtypeStruct((2, ACC_ROWS, HID), jnp.float32),
        mesh=mesh,
        scratch_types=[
            pltpu.VMEM((IB, CH), jnp.int32),
            pltpu.VMEM((IB, CH), jnp.int32),
            pltpu.VMEM((CH, HID), jnp.float32),
            pltpu.VMEM((CH, HID), jnp.float32),
            pltpu.VMEM_SHARED((ACC_ROWS, HID), jnp.float32),
            pltpu.SemaphoreType.DMA,
            pltpu.SemaphoreType.DMA,
            pltpu.SemaphoreType.DMA,
            pltpu.SemaphoreType.DMA,
        ],
    )


def _sc_aggregate(t2, gidx3, dst3):
    return _get_sc_aggregate()(t2, gidx3, dst3)


def _mm_body(h_ref, w2_ref, ws_ref, t_ref, s_ref):
    h = h_ref[...]
    t_ref[...] = jnp.dot(h, w2_ref[...], preferred_element_type=jnp.float32,
                         precision=lax.Precision.HIGHEST)
    s_ref[...] = jnp.dot(h, ws_ref[...], preferred_element_type=jnp.float32,
                         precision=lax.Precision.HIGHEST)


_matmuls = pl.pallas_call(
    _mm_body,
    grid=(NB,),
    in_specs=[
        pl.BlockSpec((BN, HID), lambda j: (j, 0)),
        pl.BlockSpec((HID, RR * HID), lambda j: (0, 0)),
        pl.BlockSpec((HID, HID), lambda j: (0, 0)),
    ],
    out_specs=[
        pl.BlockSpec((BN, RR * HID), lambda j: (j, 0)),
        pl.BlockSpec((BN, HID), lambda j: (j, 0)),
    ],
    out_shape=[
        jax.ShapeDtypeStruct((NN, RR * HID), jnp.float32),
        jax.ShapeDtypeStruct((NN, HID), jnp.float32),
    ],
)


def _norm_body(p_ref, s_ref, b_ref, g_ref, be_ref, o_ref, pre_buf, stats):
    ph = pl.program_id(0)
    j = pl.program_id(1)

    @pl.when(ph == 0)
    def _():
        pre = p_ref[0] + p_ref[1] + s_ref[...] + b_ref[...]
        pre_buf[pl.ds(j * BN, BN)] = pre

        @pl.when(j == 0)
        def _():
            stats[...] = jnp.zeros_like(stats)

        stats[0:1, :] += jnp.sum(pre, axis=0, keepdims=True)
        stats[1:2, :] += jnp.sum(pre * pre, axis=0, keepdims=True)
        o_ref[...] = pre

    @pl.when(ph == 1)
    def _():
        @pl.when(j == 0)
        def _():
            mean = stats[0:1, :] * (1.0 / NN)
            var = stats[1:2, :] * (1.0 / NN) - mean * mean
            scale = g_ref[...] * lax.rsqrt(var + 1e-5)
            shift = be_ref[...] - mean * scale
            stats[0:1, :] = scale
            stats[1:2, :] = shift

        pre = pre_buf[pl.ds(j * BN, BN)]
        o_ref[...] = jnp.maximum(pre * stats[0:1, :] + stats[1:2, :], 0.0)


_norm = pl.pallas_call(
    _norm_body,
    grid=(2, NB),
    in_specs=[
        pl.BlockSpec((2, BN, HID), lambda p, j: (0, j, 0)),
        pl.BlockSpec((BN, HID), lambda p, j: (j, 0)),
        pl.BlockSpec((1, HID), lambda p, j: (0, 0)),
        pl.BlockSpec((1, HID), lambda p, j: (0, 0)),
        pl.BlockSpec((1, HID), lambda p, j: (0, 0)),
    ],
    out_specs=pl.BlockSpec((BN, HID), lambda p, j: (j, 0)),
    out_shape=jax.ShapeDtypeStruct((NN, HID), jnp.float32),
    scratch_shapes=[
        pltpu.VMEM((NN, HID), jnp.float32),
        pltpu.VMEM((2, HID), jnp.float32),
    ],
)


def _regroup_w(w_rel, din):
    w3 = w_rel.reshape(RR, din, HID)
    if din < HID:
        w3 = jnp.pad(w3, ((0, 0), (0, HID - din), (0, 0)))
    return w3.transpose(1, 0, 2).reshape(HID, RR * HID)


def _pad_cols(a, din):
    if din < HID:
        return jnp.pad(a, ((0, 0), (0, HID - din)))
    return a


def kernel(x, edge_index, edge_type, W_rel_0, b_0, W_self_0, gamma_0, beta_0,
           W_rel_1, b_1, W_self_1, gamma_1, beta_1, W_rel_2, b_2, W_self_2,
           gamma_2, beta_2):
    src = edge_index[0]
    dst = edge_index[1]
    gidx = src * RR + edge_type
    npad = EPAD - EE
    gidx3 = jnp.concatenate(
        [gidx, jnp.zeros((npad,), jnp.int32)]).reshape(TOTCH, CH)
    dst3 = jnp.concatenate(
        [dst, jnp.full((npad,), NN, jnp.int32)]).reshape(TOTCH, CH)

    h = _pad_cols(x, x.shape[1])
    dins = [x.shape[1], HID, HID]
    params = [
        (W_rel_0, b_0, W_self_0, gamma_0, beta_0),
        (W_rel_1, b_1, W_self_1, gamma_1, beta_1),
        (W_rel_2, b_2, W_self_2, gamma_2, beta_2),
    ]
    outs = []
    for l, (wr, b, ws, g, be) in enumerate(params):
        w2 = _regroup_w(wr, dins[l])
        wsp = jnp.pad(ws, ((0, HID - dins[l]), (0, 0))) if dins[l] < HID else ws
        t2, s = _matmuls(h, w2, wsp)
        part = _sc_aggregate(t2.reshape(NN * RR, HID), gidx3, dst3)
        h = _norm(part, s, b.reshape(1, HID), g.reshape(1, HID),
                  be.reshape(1, HID))
        outs.append(h)
    return jnp.concatenate(outs, axis=-1)
